# Initial kernel scaffold; baseline (speedup 1.0000x reference)
import jax
import jax.numpy as jnp
from jax.experimental import pallas as pl


def kernel(inputs, edge_index, W1, att_src1, att_dst1, b1, W2, att_src2, att_dst2, b2):
    raise NotImplementedError("write your pallas kernel here")


# speedup vs baseline: 33.0619x; 33.0619x over previous
import functools

import jax
import jax.numpy as jnp
from jax import lax
from jax.experimental import pallas as pl
from jax.experimental.pallas import tpu as pltpu
from jax.experimental.pallas import tpu_sc as plsc

N = 10000
E = 320000
D_IN = 128
HEADS = 8
HID = 16
F1 = HEADS * HID
NCLS = 64
EPS = 1e-16

NC = 2
NS = 16
NW = NC * NS
EPW = E // NW
CH = 80
NCHUNK = EPW // CH
ROWS_PT = N // NS
ZCH = 125

BN = 1000
GRID = N // BN


def _tc1_body(x_ref, w_ref, as_ref, ad_ref, h_ref, asrc_ref, adst_ref):
    xb = x_ref[...]
    h = jnp.dot(xb, w_ref[...], preferred_element_type=jnp.float32)
    h_ref[...] = h
    cols_s = []
    cols_d = []
    for hh in range(HEADS):
        seg = h[:, hh * HID:(hh + 1) * HID]
        cols_s.append(jnp.sum(seg * as_ref[hh, :][None, :], axis=1, keepdims=True))
        cols_d.append(jnp.sum(seg * ad_ref[hh, :][None, :], axis=1, keepdims=True))
    zpad = jnp.zeros((xb.shape[0], 16 - HEADS), jnp.float32)
    asrc_ref[...] = jnp.concatenate(cols_s + [zpad], axis=1)
    adst_ref[...] = jnp.concatenate(cols_d + [zpad], axis=1)


def _tc1(x, w1, att_src1, att_dst1):
    return pl.pallas_call(
        _tc1_body,
        grid=(GRID,),
        in_specs=[
            pl.BlockSpec((BN, D_IN), lambda i: (i, 0)),
            pl.BlockSpec((D_IN, F1), lambda i: (0, 0)),
            pl.BlockSpec((HEADS, HID), lambda i: (0, 0)),
            pl.BlockSpec((HEADS, HID), lambda i: (0, 0)),
        ],
        out_specs=[
            pl.BlockSpec((BN, F1), lambda i: (i, 0)),
            pl.BlockSpec((BN, 16), lambda i: (i, 0)),
            pl.BlockSpec((BN, 16), lambda i: (i, 0)),
        ],
        out_shape=[
            jax.ShapeDtypeStruct((N, F1), jnp.float32),
            jax.ShapeDtypeStruct((N, 16), jnp.float32),
            jax.ShapeDtypeStruct((N, 16), jnp.float32),
        ],
    )(x, w1, att_src1, att_dst1)


def _tc2_body(p0_ref, p1_ref, d0_ref, d1_ref, w_ref, b1_ref, as_ref, ad_ref,
              h2_ref, asrc_ref, adst_ref):
    s = p0_ref[...] + p1_ref[...]
    den = d0_ref[...] + d1_ref[...]
    parts = []
    for hh in range(HEADS):
        dh = den[:, hh:hh + 1] + EPS
        parts.append(s[:, hh * HID:(hh + 1) * HID] / dh)
    x2 = jnp.concatenate(parts, axis=1) + b1_ref[...]
    x2 = jnp.maximum(x2, 0.0)
    h2 = jnp.dot(x2, w_ref[...], preferred_element_type=jnp.float32)
    h2_ref[...] = h2
    zpad = jnp.zeros((x2.shape[0], 15), jnp.float32)
    asrc_ref[...] = jnp.concatenate(
        [jnp.sum(h2 * as_ref[...], axis=1, keepdims=True), zpad], axis=1)
    adst_ref[...] = jnp.concatenate(
        [jnp.sum(h2 * ad_ref[...], axis=1, keepdims=True), zpad], axis=1)


def _tc2(p0, p1, d0, d1, w2, b1, att_src2, att_dst2):
    return pl.pallas_call(
        _tc2_body,
        grid=(GRID,),
        in_specs=[
            pl.BlockSpec((BN, F1), lambda i: (i, 0)),
            pl.BlockSpec((BN, F1), lambda i: (i, 0)),
            pl.BlockSpec((BN, 16), lambda i: (i, 0)),
            pl.BlockSpec((BN, 16), lambda i: (i, 0)),
            pl.BlockSpec((F1, NCLS), lambda i: (0, 0)),
            pl.BlockSpec((1, F1), lambda i: (0, 0)),
            pl.BlockSpec((1, NCLS), lambda i: (0, 0)),
            pl.BlockSpec((1, NCLS), lambda i: (0, 0)),
        ],
        out_specs=[
            pl.BlockSpec((BN, NCLS), lambda i: (i, 0)),
            pl.BlockSpec((BN, 16), lambda i: (i, 0)),
            pl.BlockSpec((BN, 16), lambda i: (i, 0)),
        ],
        out_shape=[
            jax.ShapeDtypeStruct((N, NCLS), jnp.float32),
            jax.ShapeDtypeStruct((N, 16), jnp.float32),
            jax.ShapeDtypeStruct((N, 16), jnp.float32),
        ],
    )(p0, p1, d0, d1, w2, b1, att_src2, att_dst2)


def _tc3_body(q0_ref, q1_ref, e0_ref, e1_ref, b2_ref, out_ref):
    s = q0_ref[...] + q1_ref[...]
    den = e0_ref[...][:, 0:1] + e1_ref[...][:, 0:1] + EPS
    out_ref[...] = s / den + b2_ref[...]


def _tc3(q0, q1, e0, e1, b2):
    return pl.pallas_call(
        _tc3_body,
        grid=(GRID,),
        in_specs=[
            pl.BlockSpec((BN, NCLS), lambda i: (i, 0)),
            pl.BlockSpec((BN, NCLS), lambda i: (i, 0)),
            pl.BlockSpec((BN, 16), lambda i: (i, 0)),
            pl.BlockSpec((BN, 16), lambda i: (i, 0)),
            pl.BlockSpec((1, NCLS), lambda i: (0, 0)),
        ],
        out_specs=pl.BlockSpec((BN, NCLS), lambda i: (i, 0)),
        out_shape=jax.ShapeDtypeStruct((N, NCLS), jnp.float32),
    )(q0, q1, e0, e1, b2)


@functools.lru_cache(maxsize=None)
def _make_sc_layer(F, heads):
    nq = F // 16
    mesh = plsc.VectorSubcoreMesh(core_axis_name="c", subcore_axis_name="s")

    @functools.partial(
        pl.kernel,
        out_type=(
            jax.ShapeDtypeStruct((NC, N, F), jnp.float32),
            jax.ShapeDtypeStruct((NC, N, 16), jnp.float32),
        ),
        mesh=mesh,
        compiler_params=pltpu.CompilerParams(use_tc_tiling_on_sc=False),
        scratch_types=[
            pltpu.VMEM((CH,), jnp.int32),
            pltpu.VMEM((CH,), jnp.int32),
            pltpu.VMEM((CH, 16), jnp.float32),
            pltpu.VMEM((CH, 16), jnp.float32),
            pltpu.VMEM((CH, F), jnp.float32),
            pltpu.VMEM((ROWS_PT, 16), jnp.float32),
            pltpu.VMEM((ZCH, F), jnp.float32),
            pltpu.VMEM_SHARED((N, F), jnp.float32),
            pltpu.VMEM_SHARED((N, 16), jnp.float32),
        ],
    )
    def sck(h_hbm, asrc_hbm, adst_hbm, src_hbm, dst_hbm,
            outp_hbm, denp_hbm,
            srcv, dstv, av, bv, hv, dbuf, fbuf, out_sh, den_sh):
        cid = lax.axis_index("c")
        sid = lax.axis_index("s")
        wid = sid * NC + cid
        tbase = sid * ROWS_PT

        def zrow_f(r, carry):
            for q in range(nq):
                fbuf[r, pl.ds(q * 16, 16)] = jnp.zeros((16,), jnp.float32)
            return carry

        lax.fori_loop(0, ZCH, zrow_f, 0)

        def zrow_d(r, carry):
            dbuf[r, :] = jnp.zeros((16,), jnp.float32)
            return carry

        lax.fori_loop(0, ROWS_PT, zrow_d, 0)
        for k in range(ROWS_PT // ZCH):
            pltpu.sync_copy(fbuf, out_sh.at[pl.ds(tbase + k * ZCH, ZCH)])
        pltpu.sync_copy(dbuf, den_sh.at[pl.ds(tbase, ROWS_PT)])
        plsc.subcore_barrier()

        ebase = wid * EPW

        def chunk_body(i, carry):
            off = pl.multiple_of(ebase + i * CH, 8)
            pltpu.sync_copy(src_hbm.at[pl.ds(off, CH)], srcv)
            pltpu.sync_copy(dst_hbm.at[pl.ds(off, CH)], dstv)
            pltpu.sync_copy(asrc_hbm.at[srcv], av)
            pltpu.sync_copy(adst_hbm.at[dstv], bv)
            pltpu.sync_copy(h_hbm.at[srcv], hv)

            def edge_body(c, inner):
                al = av[c, :] + bv[c, :]
                al = jnp.where(al > 0.0, al, 0.2 * al)
                exv = jnp.exp(al)
                bv[c, :] = exv
                for q in range(nq):
                    sc = exv[q * heads // nq]
                    hv[c, pl.ds(q * 16, 16)] = hv[c, pl.ds(q * 16, 16)] * sc
                return inner

            lax.fori_loop(0, CH, edge_body, 0)
            pltpu.sync_copy(bv, den_sh.at[dstv], add=True)
            pltpu.sync_copy(hv, out_sh.at[dstv], add=True)
            return carry

        lax.fori_loop(0, NCHUNK, chunk_body, 0)
        plsc.subcore_barrier()

        for k in range(ROWS_PT // ZCH):
            pltpu.sync_copy(out_sh.at[pl.ds(tbase + k * ZCH, ZCH)], fbuf)
            pltpu.sync_copy(fbuf, outp_hbm.at[cid, pl.ds(tbase + k * ZCH, ZCH)])
        pltpu.sync_copy(den_sh.at[pl.ds(tbase, ROWS_PT)], dbuf)
        pltpu.sync_copy(dbuf, denp_hbm.at[cid, pl.ds(tbase, ROWS_PT)])

    return sck


def kernel(inputs, edge_index, W1, att_src1, att_dst1, b1,
           W2, att_src2, att_dst2, b2):
    src = edge_index[0]
    dst = edge_index[1]

    h1, asrc1, adst1 = _tc1(inputs, W1, att_src1, att_dst1)
    outp1, denp1 = _make_sc_layer(F1, HEADS)(h1, asrc1, adst1, src, dst)

    h2, asrc2, adst2 = _tc2(outp1[0], outp1[1], denp1[0], denp1[1],
                            W2, b1.reshape(1, F1), att_src2, att_dst2)
    outp2, denp2 = _make_sc_layer(NCLS, 1)(h2, asrc2, adst2, src, dst)

    return _tc3(outp2[0], outp2[1], denp2[0], denp2[1], b2.reshape(1, NCLS))


# speedup vs baseline: 47.1156x; 1.4251x over previous
import functools

import jax
import jax.numpy as jnp
from jax import lax
from jax.experimental import pallas as pl
from jax.experimental.pallas import tpu as pltpu
from jax.experimental.pallas import tpu_sc as plsc

N = 10000
E = 320000
D_IN = 128
HEADS = 8
HID = 16
F1 = HEADS * HID
NCLS = 64
EPS = 1e-16

NC = 2
NS = 16
NW = NC * NS
EPW = E // NW
CH = 80
NCHUNK = EPW // CH
ROWS_PT = N // NS
ZCH = 125

BN = 1000
GRID = N // BN


def _tc1_body(x_ref, w_ref, as_ref, ad_ref, h_ref, asrc_ref, adst_ref):
    xb = x_ref[...]
    h = jnp.dot(xb, w_ref[...], preferred_element_type=jnp.float32)
    h_ref[...] = h
    cols_s = []
    cols_d = []
    for hh in range(HEADS):
        seg = h[:, hh * HID:(hh + 1) * HID]
        cols_s.append(jnp.sum(seg * as_ref[hh, :][None, :], axis=1, keepdims=True))
        cols_d.append(jnp.sum(seg * ad_ref[hh, :][None, :], axis=1, keepdims=True))
    zpad = jnp.zeros((xb.shape[0], 16 - HEADS), jnp.float32)
    asrc_ref[...] = jnp.concatenate(cols_s + [zpad], axis=1)
    adst_ref[...] = jnp.concatenate(cols_d + [zpad], axis=1)


def _tc1(x, w1, att_src1, att_dst1):
    return pl.pallas_call(
        _tc1_body,
        grid=(GRID,),
        in_specs=[
            pl.BlockSpec((BN, D_IN), lambda i: (i, 0)),
            pl.BlockSpec((D_IN, F1), lambda i: (0, 0)),
            pl.BlockSpec((HEADS, HID), lambda i: (0, 0)),
            pl.BlockSpec((HEADS, HID), lambda i: (0, 0)),
        ],
        out_specs=[
            pl.BlockSpec((BN, F1), lambda i: (i, 0)),
            pl.BlockSpec((BN, 16), lambda i: (i, 0)),
            pl.BlockSpec((BN, 16), lambda i: (i, 0)),
        ],
        out_shape=[
            jax.ShapeDtypeStruct((N, F1), jnp.float32),
            jax.ShapeDtypeStruct((N, 16), jnp.float32),
            jax.ShapeDtypeStruct((N, 16), jnp.float32),
        ],
    )(x, w1, att_src1, att_dst1)


def _tc2_body(p0_ref, p1_ref, d0_ref, d1_ref, w_ref, b1_ref, as_ref, ad_ref,
              h2_ref, asrc_ref, adst_ref):
    s = p0_ref[...] + p1_ref[...]
    den = d0_ref[...] + d1_ref[...]
    parts = []
    for hh in range(HEADS):
        dh = den[:, hh:hh + 1] + EPS
        parts.append(s[:, hh * HID:(hh + 1) * HID] / dh)
    x2 = jnp.concatenate(parts, axis=1) + b1_ref[...]
    x2 = jnp.maximum(x2, 0.0)
    h2 = jnp.dot(x2, w_ref[...], preferred_element_type=jnp.float32)
    h2_ref[...] = h2
    zpad = jnp.zeros((x2.shape[0], 15), jnp.float32)
    asrc_ref[...] = jnp.concatenate(
        [jnp.sum(h2 * as_ref[...], axis=1, keepdims=True), zpad], axis=1)
    adst_ref[...] = jnp.concatenate(
        [jnp.sum(h2 * ad_ref[...], axis=1, keepdims=True), zpad], axis=1)


def _tc2(p0, p1, d0, d1, w2, b1, att_src2, att_dst2):
    return pl.pallas_call(
        _tc2_body,
        grid=(GRID,),
        in_specs=[
            pl.BlockSpec((BN, F1), lambda i: (i, 0)),
            pl.BlockSpec((BN, F1), lambda i: (i, 0)),
            pl.BlockSpec((BN, 16), lambda i: (i, 0)),
            pl.BlockSpec((BN, 16), lambda i: (i, 0)),
            pl.BlockSpec((F1, NCLS), lambda i: (0, 0)),
            pl.BlockSpec((1, F1), lambda i: (0, 0)),
            pl.BlockSpec((1, NCLS), lambda i: (0, 0)),
            pl.BlockSpec((1, NCLS), lambda i: (0, 0)),
        ],
        out_specs=[
            pl.BlockSpec((BN, NCLS), lambda i: (i, 0)),
            pl.BlockSpec((BN, 16), lambda i: (i, 0)),
            pl.BlockSpec((BN, 16), lambda i: (i, 0)),
        ],
        out_shape=[
            jax.ShapeDtypeStruct((N, NCLS), jnp.float32),
            jax.ShapeDtypeStruct((N, 16), jnp.float32),
            jax.ShapeDtypeStruct((N, 16), jnp.float32),
        ],
    )(p0, p1, d0, d1, w2, b1, att_src2, att_dst2)


def _tc3_body(q0_ref, q1_ref, e0_ref, e1_ref, b2_ref, out_ref):
    s = q0_ref[...] + q1_ref[...]
    den = e0_ref[...][:, 0:1] + e1_ref[...][:, 0:1] + EPS
    out_ref[...] = s / den + b2_ref[...]


def _tc3(q0, q1, e0, e1, b2):
    return pl.pallas_call(
        _tc3_body,
        grid=(GRID,),
        in_specs=[
            pl.BlockSpec((BN, NCLS), lambda i: (i, 0)),
            pl.BlockSpec((BN, NCLS), lambda i: (i, 0)),
            pl.BlockSpec((BN, 16), lambda i: (i, 0)),
            pl.BlockSpec((BN, 16), lambda i: (i, 0)),
            pl.BlockSpec((1, NCLS), lambda i: (0, 0)),
        ],
        out_specs=pl.BlockSpec((BN, NCLS), lambda i: (i, 0)),
        out_shape=jax.ShapeDtypeStruct((N, NCLS), jnp.float32),
    )(q0, q1, e0, e1, b2)


@functools.lru_cache(maxsize=None)
def _make_sc_layer(F, heads):
    nq = F // 16
    mesh = plsc.VectorSubcoreMesh(core_axis_name="c", subcore_axis_name="s")

    @functools.partial(
        pl.kernel,
        out_type=(
            jax.ShapeDtypeStruct((NC, N, F), jnp.float32),
            jax.ShapeDtypeStruct((NC, N, 16), jnp.float32),
        ),
        mesh=mesh,
        compiler_params=pltpu.CompilerParams(use_tc_tiling_on_sc=False),
        scratch_types=[
            pltpu.VMEM((CH,), jnp.int32),
            pltpu.VMEM((CH,), jnp.int32),
            pltpu.VMEM((CH, 16), jnp.float32),
            pltpu.VMEM((CH, 16), jnp.float32),
            pltpu.VMEM((CH, F), jnp.float32),
            pltpu.VMEM((ROWS_PT, 16), jnp.float32),
            pltpu.VMEM((ZCH, F), jnp.float32),
            pltpu.VMEM_SHARED((N, F), jnp.float32),
            pltpu.VMEM_SHARED((N, 16), jnp.float32),
            pltpu.SemaphoreType.DMA,
            pltpu.SemaphoreType.DMA,
            pltpu.SemaphoreType.DMA,
        ],
    )
    def sck(h_hbm, asrc_hbm, adst_hbm, src_hbm, dst_hbm,
            outp_hbm, denp_hbm,
            srcv, dstv, av, bv, hv, dbuf, fbuf, out_sh, den_sh,
            gsem, hsem, ssem):
        cid = lax.axis_index("c")
        sid = lax.axis_index("s")
        wid = sid * NC + cid
        tbase = sid * ROWS_PT

        def zrow_f(r, carry):
            for q in range(nq):
                fbuf[r, pl.ds(q * 16, 16)] = jnp.zeros((16,), jnp.float32)
            return carry

        lax.fori_loop(0, ZCH, zrow_f, 0)

        def zrow_d(r, carry):
            dbuf[r, :] = jnp.zeros((16,), jnp.float32)
            return carry

        lax.fori_loop(0, ROWS_PT, zrow_d, 0)
        for k in range(ROWS_PT // ZCH):
            pltpu.sync_copy(fbuf, out_sh.at[pl.ds(tbase + k * ZCH, ZCH)])
        pltpu.sync_copy(dbuf, den_sh.at[pl.ds(tbase, ROWS_PT)])
        plsc.subcore_barrier()

        ebase = wid * EPW

        def chunk_body(i, carry):
            off = pl.multiple_of(ebase + i * CH, 8)
            i1 = pltpu.async_copy(src_hbm.at[pl.ds(off, CH)], srcv, gsem)
            i2 = pltpu.async_copy(dst_hbm.at[pl.ds(off, CH)], dstv, gsem)
            i1.wait()
            i2.wait()
            g3 = pltpu.async_copy(h_hbm.at[srcv], hv, hsem)
            g1 = pltpu.async_copy(asrc_hbm.at[srcv], av, gsem)
            g2 = pltpu.async_copy(adst_hbm.at[dstv], bv, gsem)
            g1.wait()
            g2.wait()

            def ex_body(c, inner):
                al = av[c, :] + bv[c, :]
                al = jnp.where(al > 0.0, al, 0.2 * al)
                bv[c, :] = jnp.exp(al)
                return inner

            lax.fori_loop(0, CH, ex_body, 0)
            s1 = pltpu.async_copy(bv, den_sh.at[dstv], ssem, add=True)
            g3.wait()

            def scale_body(c, inner):
                exv = bv[c, :]
                for q in range(nq):
                    sc = exv[q * heads // nq]
                    hv[c, pl.ds(q * 16, 16)] = hv[c, pl.ds(q * 16, 16)] * sc
                return inner

            lax.fori_loop(0, CH, scale_body, 0)
            s2 = pltpu.async_copy(hv, out_sh.at[dstv], ssem, add=True)
            s1.wait()
            s2.wait()
            return carry

        lax.fori_loop(0, NCHUNK, chunk_body, 0)
        plsc.subcore_barrier()

        for k in range(ROWS_PT // ZCH):
            pltpu.sync_copy(out_sh.at[pl.ds(tbase + k * ZCH, ZCH)], fbuf)
            pltpu.sync_copy(fbuf, outp_hbm.at[cid, pl.ds(tbase + k * ZCH, ZCH)])
        pltpu.sync_copy(den_sh.at[pl.ds(tbase, ROWS_PT)], dbuf)
        pltpu.sync_copy(dbuf, denp_hbm.at[cid, pl.ds(tbase, ROWS_PT)])

    return sck


def kernel(inputs, edge_index, W1, att_src1, att_dst1, b1,
           W2, att_src2, att_dst2, b2):
    src = edge_index[0]
    dst = edge_index[1]

    h1, asrc1, adst1 = _tc1(inputs, W1, att_src1, att_dst1)
    outp1, denp1 = _make_sc_layer(F1, HEADS)(h1, asrc1, adst1, src, dst)

    h2, asrc2, adst2 = _tc2(outp1[0], outp1[1], denp1[0], denp1[1],
                            W2, b1.reshape(1, F1), att_src2, att_dst2)
    outp2, denp2 = _make_sc_layer(NCLS, 1)(h2, asrc2, adst2, src, dst)

    return _tc3(outp2[0], outp2[1], denp2[0], denp2[1], b2.reshape(1, NCLS))


# speedup vs baseline: 47.6943x; 1.0123x over previous
import functools

import jax
import jax.numpy as jnp
from jax import lax
from jax.experimental import pallas as pl
from jax.experimental.pallas import tpu as pltpu
from jax.experimental.pallas import tpu_sc as plsc

N = 10000
E = 320000
D_IN = 128
HEADS = 8
HID = 16
F1 = HEADS * HID
NCLS = 64
EPS = 1e-16

NC = 2
NS = 16
NW = NC * NS
EPW = E // NW
CH = 80
NCHUNK = EPW // CH
ROWS_PT = N // NS
ZCH = 125

BN = 1000
GRID = N // BN


def _tc1_body(x_ref, w_ref, as_ref, ad_ref, h_ref, asrc_ref, adst_ref):
    xb = x_ref[...]
    h = jnp.dot(xb, w_ref[...], preferred_element_type=jnp.float32)
    h_ref[...] = h
    cols_s = []
    cols_d = []
    for hh in range(HEADS):
        seg = h[:, hh * HID:(hh + 1) * HID]
        cols_s.append(jnp.sum(seg * as_ref[hh, :][None, :], axis=1, keepdims=True))
        cols_d.append(jnp.sum(seg * ad_ref[hh, :][None, :], axis=1, keepdims=True))
    zpad = jnp.zeros((xb.shape[0], 16 - HEADS), jnp.float32)
    asrc_ref[...] = jnp.concatenate(cols_s + [zpad], axis=1)
    adst_ref[...] = jnp.concatenate(cols_d + [zpad], axis=1)


def _tc1(x, w1, att_src1, att_dst1):
    return pl.pallas_call(
        _tc1_body,
        grid=(GRID,),
        in_specs=[
            pl.BlockSpec((BN, D_IN), lambda i: (i, 0)),
            pl.BlockSpec((D_IN, F1), lambda i: (0, 0)),
            pl.BlockSpec((HEADS, HID), lambda i: (0, 0)),
            pl.BlockSpec((HEADS, HID), lambda i: (0, 0)),
        ],
        out_specs=[
            pl.BlockSpec((BN, F1), lambda i: (i, 0)),
            pl.BlockSpec((BN, 16), lambda i: (i, 0)),
            pl.BlockSpec((BN, 16), lambda i: (i, 0)),
        ],
        out_shape=[
            jax.ShapeDtypeStruct((N, F1), jnp.float32),
            jax.ShapeDtypeStruct((N, 16), jnp.float32),
            jax.ShapeDtypeStruct((N, 16), jnp.float32),
        ],
    )(x, w1, att_src1, att_dst1)


def _tc2_body(p0_ref, p1_ref, d0_ref, d1_ref, w_ref, b1_ref, as_ref, ad_ref,
              h2_ref, asrc_ref, adst_ref):
    s = p0_ref[...] + p1_ref[...]
    den = d0_ref[...] + d1_ref[...]
    parts = []
    for hh in range(HEADS):
        dh = den[:, hh:hh + 1] + EPS
        parts.append(s[:, hh * HID:(hh + 1) * HID] / dh)
    x2 = jnp.concatenate(parts, axis=1) + b1_ref[...]
    x2 = jnp.maximum(x2, 0.0)
    h2 = jnp.dot(x2, w_ref[...], preferred_element_type=jnp.float32)
    h2_ref[...] = h2
    zpad = jnp.zeros((x2.shape[0], 15), jnp.float32)
    asrc_ref[...] = jnp.concatenate(
        [jnp.sum(h2 * as_ref[...], axis=1, keepdims=True), zpad], axis=1)
    adst_ref[...] = jnp.concatenate(
        [jnp.sum(h2 * ad_ref[...], axis=1, keepdims=True), zpad], axis=1)


def _tc2(p0, p1, d0, d1, w2, b1, att_src2, att_dst2):
    return pl.pallas_call(
        _tc2_body,
        grid=(GRID,),
        in_specs=[
            pl.BlockSpec((BN, F1), lambda i: (i, 0)),
            pl.BlockSpec((BN, F1), lambda i: (i, 0)),
            pl.BlockSpec((BN, 16), lambda i: (i, 0)),
            pl.BlockSpec((BN, 16), lambda i: (i, 0)),
            pl.BlockSpec((F1, NCLS), lambda i: (0, 0)),
            pl.BlockSpec((1, F1), lambda i: (0, 0)),
            pl.BlockSpec((1, NCLS), lambda i: (0, 0)),
            pl.BlockSpec((1, NCLS), lambda i: (0, 0)),
        ],
        out_specs=[
            pl.BlockSpec((BN, NCLS), lambda i: (i, 0)),
            pl.BlockSpec((BN, 16), lambda i: (i, 0)),
            pl.BlockSpec((BN, 16), lambda i: (i, 0)),
        ],
        out_shape=[
            jax.ShapeDtypeStruct((N, NCLS), jnp.float32),
            jax.ShapeDtypeStruct((N, 16), jnp.float32),
            jax.ShapeDtypeStruct((N, 16), jnp.float32),
        ],
    )(p0, p1, d0, d1, w2, b1, att_src2, att_dst2)


def _tc3_body(q0_ref, q1_ref, e0_ref, e1_ref, b2_ref, out_ref):
    s = q0_ref[...] + q1_ref[...]
    den = e0_ref[...][:, 0:1] + e1_ref[...][:, 0:1] + EPS
    out_ref[...] = s / den + b2_ref[...]


def _tc3(q0, q1, e0, e1, b2):
    return pl.pallas_call(
        _tc3_body,
        grid=(GRID,),
        in_specs=[
            pl.BlockSpec((BN, NCLS), lambda i: (i, 0)),
            pl.BlockSpec((BN, NCLS), lambda i: (i, 0)),
            pl.BlockSpec((BN, 16), lambda i: (i, 0)),
            pl.BlockSpec((BN, 16), lambda i: (i, 0)),
            pl.BlockSpec((1, NCLS), lambda i: (0, 0)),
        ],
        out_specs=pl.BlockSpec((BN, NCLS), lambda i: (i, 0)),
        out_shape=jax.ShapeDtypeStruct((N, NCLS), jnp.float32),
    )(q0, q1, e0, e1, b2)


@functools.lru_cache(maxsize=None)
def _make_sc_layer(F, heads):
    nq = F // 16
    mesh = plsc.VectorSubcoreMesh(core_axis_name="c", subcore_axis_name="s")

    @functools.partial(
        pl.kernel,
        out_type=(
            jax.ShapeDtypeStruct((NC, N, F), jnp.float32),
            jax.ShapeDtypeStruct((NC, N, 16), jnp.float32),
        ),
        mesh=mesh,
        compiler_params=pltpu.CompilerParams(use_tc_tiling_on_sc=False),
        scratch_types=[
            pltpu.VMEM((CH,), jnp.int32),
            pltpu.VMEM((CH,), jnp.int32),
            pltpu.VMEM((CH, 16), jnp.float32),
            pltpu.VMEM((CH, 16), jnp.float32),
            pltpu.VMEM((CH, F), jnp.float32),
            pltpu.VMEM((ROWS_PT, 16), jnp.float32),
            pltpu.VMEM((ZCH, F), jnp.float32),
            pltpu.VMEM_SHARED((N, F), jnp.float32),
            pltpu.VMEM_SHARED((N, 16), jnp.float32),
            pltpu.SemaphoreType.DMA,
            pltpu.SemaphoreType.DMA,
            pltpu.SemaphoreType.DMA,
        ],
    )
    def sck(h_hbm, asrc_hbm, adst_hbm, src_hbm, dst_hbm,
            outp_hbm, denp_hbm,
            srcv, dstv, av, bv, hv, dbuf, fbuf, out_sh, den_sh,
            gsem, hsem, ssem):
        cid = lax.axis_index("c")
        sid = lax.axis_index("s")
        wid = sid * NC + cid
        tbase = sid * ROWS_PT

        def zrow_f(r, carry):
            for q in range(nq):
                fbuf[r, pl.ds(q * 16, 16)] = jnp.zeros((16,), jnp.float32)
            return carry

        lax.fori_loop(0, ZCH, zrow_f, 0)

        def zrow_d(r, carry):
            dbuf[r, :] = jnp.zeros((16,), jnp.float32)
            return carry

        lax.fori_loop(0, ROWS_PT, zrow_d, 0)
        for k in range(ROWS_PT // ZCH):
            pltpu.sync_copy(fbuf, out_sh.at[pl.ds(tbase + k * ZCH, ZCH)])
        pltpu.sync_copy(dbuf, den_sh.at[pl.ds(tbase, ROWS_PT)])
        plsc.subcore_barrier()

        ebase = wid * EPW

        def chunk_body(i, carry):
            off = pl.multiple_of(ebase + i * CH, 8)
            i1 = pltpu.async_copy(src_hbm.at[pl.ds(off, CH)], srcv, gsem)
            i2 = pltpu.async_copy(dst_hbm.at[pl.ds(off, CH)], dstv, gsem)
            i1.wait()
            i2.wait()
            g3 = pltpu.async_copy(h_hbm.at[srcv], hv, hsem)
            g1 = pltpu.async_copy(asrc_hbm.at[srcv], av, gsem)
            g2 = pltpu.async_copy(adst_hbm.at[dstv], bv, gsem)
            g1.wait()
            g2.wait()

            def ex_body(c, inner):
                al = av[c, :] + bv[c, :]
                al = jnp.where(al > 0.0, al, 0.2 * al)
                bv[c, :] = jnp.exp(al)
                return inner

            lax.fori_loop(0, CH, ex_body, 0, unroll=4)
            s1 = pltpu.async_copy(bv, den_sh.at[dstv], ssem, add=True)
            g3.wait()

            def scale_body(c, inner):
                exv = bv[c, :]
                for q in range(nq):
                    sc = exv[q * heads // nq]
                    hv[c, pl.ds(q * 16, 16)] = hv[c, pl.ds(q * 16, 16)] * sc
                return inner

            lax.fori_loop(0, CH, scale_body, 0, unroll=4)
            s2 = pltpu.async_copy(hv, out_sh.at[dstv], ssem, add=True)
            s1.wait()
            s2.wait()
            return carry

        lax.fori_loop(0, NCHUNK, chunk_body, 0)
        plsc.subcore_barrier()

        for k in range(ROWS_PT // ZCH):
            pltpu.sync_copy(out_sh.at[pl.ds(tbase + k * ZCH, ZCH)], fbuf)
            pltpu.sync_copy(fbuf, outp_hbm.at[cid, pl.ds(tbase + k * ZCH, ZCH)])
        pltpu.sync_copy(den_sh.at[pl.ds(tbase, ROWS_PT)], dbuf)
        pltpu.sync_copy(dbuf, denp_hbm.at[cid, pl.ds(tbase, ROWS_PT)])

    return sck


def kernel(inputs, edge_index, W1, att_src1, att_dst1, b1,
           W2, att_src2, att_dst2, b2):
    src = edge_index[0]
    dst = edge_index[1]

    h1, asrc1, adst1 = _tc1(inputs, W1, att_src1, att_dst1)
    outp1, denp1 = _make_sc_layer(F1, HEADS)(h1, asrc1, adst1, src, dst)

    h2, asrc2, adst2 = _tc2(outp1[0], outp1[1], denp1[0], denp1[1],
                            W2, b1.reshape(1, F1), att_src2, att_dst2)
    outp2, denp2 = _make_sc_layer(NCLS, 1)(h2, asrc2, adst2, src, dst)

    return _tc3(outp2[0], outp2[1], denp2[0], denp2[1], b2.reshape(1, NCLS))
